# baseline (speedup 1.0000x reference)
import functools

import jax
import jax.numpy as jnp
from jax import lax
from jax.experimental import pallas as pl
from jax.experimental.pallas import tpu as pltpu
from jax.experimental.pallas import tpu_sc as plsc

T = 2048
D = 2048
FF = 1024
E = 8
TM = 512
CAP = 2 * T + E * TM
NBLK = CAP // TM

_INTERPRET = False


def _router_plan_body(x_ref, gw_ref, logits_ref, wp_ref, pos_ref, be_ref):
    x = x_ref[...]
    gw = gw_ref[...]
    logits = jnp.dot(x, gw, preferred_element_type=jnp.float32)
    logits_ref[...] = logits

    m = jnp.max(logits, axis=1, keepdims=True)
    p = jnp.exp(logits - m)
    p = p / jnp.sum(p, axis=1, keepdims=True)

    lane = jax.lax.broadcasted_iota(jnp.int32, (T, E), 1)
    m0 = jnp.max(p, axis=1, keepdims=True)
    a0 = jnp.min(jnp.where(p == m0, lane, E), axis=1, keepdims=True)
    p1 = jnp.where(lane == a0, -1.0, p)
    m1 = jnp.max(p1, axis=1, keepdims=True)
    a1 = jnp.min(jnp.where(p1 == m1, lane, E), axis=1, keepdims=True)
    s = m0 + m1
    wp_ref[:, 0:1] = m0 / s
    wp_ref[:, 1:2] = m1 / s

    oh0 = (lane == a0).astype(jnp.float32)
    oh1 = (lane == a1).astype(jnp.float32)

    counts = jnp.sum(oh0, axis=0, keepdims=True) + jnp.sum(oh1, axis=0, keepdims=True)
    padded = jnp.floor((counts + (TM - 1)) / TM) * TM
    r8 = jax.lax.broadcasted_iota(jnp.int32, (E, E), 0)
    c8 = jax.lax.broadcasted_iota(jnp.int32, (E, E), 1)
    mstrict = (r8 < c8).astype(jnp.float32)
    offs = jnp.dot(padded, mstrict, preferred_element_type=jnp.float32)

    B = 512
    rb = jax.lax.broadcasted_iota(jnp.int32, (B, B), 0)
    cb = jax.lax.broadcasted_iota(jnp.int32, (B, B), 1)
    ltri = (rb >= cb).astype(jnp.float32)
    carry = jnp.zeros((1, E), jnp.float32)
    excl_parts = []
    for half in (oh0, oh1):
        for b in range(T // B):
            blk = half[b * B:(b + 1) * B, :]
            incl = jnp.dot(ltri, blk, preferred_element_type=jnp.float32) + carry
            excl_parts.append(incl - blk)
            carry = incl[B - 1:B, :]
    oh = jnp.concatenate([oh0, oh1], axis=0)
    excl = jnp.concatenate(excl_parts, axis=0)
    posf = jnp.sum(oh * (offs + excl), axis=1, keepdims=True)
    pos_ref[...] = posf.astype(jnp.int32)

    incl_off = offs + padded
    jv = (jax.lax.broadcasted_iota(jnp.int32, (NBLK, E), 0) * TM).astype(jnp.float32)
    be = jnp.sum((incl_off <= jv).astype(jnp.float32), axis=1, keepdims=True)
    used = jv[:, 0:1] < incl_off[:, E - 1:E]
    be_ref[...] = jnp.where(used, be, -1.0).astype(jnp.int32)


def _router_plan(x, gate_w):
    return pl.pallas_call(
        _router_plan_body,
        out_shape=[
            jax.ShapeDtypeStruct((T, E), jnp.float32),
            jax.ShapeDtypeStruct((T, 2), jnp.float32),
            jax.ShapeDtypeStruct((2 * T, 1), jnp.int32),
            jax.ShapeDtypeStruct((NBLK, 1), jnp.int32),
        ],
        interpret=_INTERPRET,
    )(x, gate_w)


def _gmm1_body(be_sref, xs_ref, w1g_ref, w1u_ref, a_ref):
    @pl.when(be_sref[pl.program_id(0)] >= 0)
    def _():
        x = xs_ref[...]
        g = jnp.dot(x, w1g_ref[0], preferred_element_type=jnp.float32)
        u = jnp.dot(x, w1u_ref[0], preferred_element_type=jnp.float32)
        a_ref[...] = g * jax.nn.sigmoid(g) * u


def _gmm1(xs, w1, be):
    grid_spec = pltpu.PrefetchScalarGridSpec(
        num_scalar_prefetch=1,
        grid=(NBLK,),
        in_specs=[
            pl.BlockSpec((TM, D), lambda i, be_ref: (i, 0)),
            pl.BlockSpec((1, D, FF), lambda i, be_ref: (jnp.maximum(be_ref[i], 0), 0, 0)),
            pl.BlockSpec((1, D, FF), lambda i, be_ref: (jnp.maximum(be_ref[i], 0), 0, 1)),
        ],
        out_specs=pl.BlockSpec((TM, FF), lambda i, be_ref: (i, 0)),
    )
    return pl.pallas_call(
        _gmm1_body,
        grid_spec=grid_spec,
        out_shape=jax.ShapeDtypeStruct((CAP, FF), jnp.float32),
        interpret=_INTERPRET,
    )(be, xs, w1, w1)


def _gmm2_body(be_sref, a_ref, w2_ref, y_ref):
    @pl.when(be_sref[pl.program_id(0)] >= 0)
    def _():
        y_ref[...] = jnp.dot(a_ref[...], w2_ref[0], preferred_element_type=jnp.float32)


def _gmm2(a, w2, be):
    grid_spec = pltpu.PrefetchScalarGridSpec(
        num_scalar_prefetch=1,
        grid=(NBLK,),
        in_specs=[
            pl.BlockSpec((TM, FF), lambda i, be_ref: (i, 0)),
            pl.BlockSpec((1, FF, D), lambda i, be_ref: (jnp.maximum(be_ref[i], 0), 0, 0)),
        ],
        out_specs=pl.BlockSpec((TM, D), lambda i, be_ref: (i, 0)),
    )
    return pl.pallas_call(
        _gmm2_body,
        grid_spec=grid_spec,
        out_shape=jax.ShapeDtypeStruct((CAP, D), jnp.float32),
        interpret=_INTERPRET,
    )(be, a, w2)


NW = 32
TPW = T // NW
SUB = 16
NSUB = TPW // SUB


def _sc_mesh():
    return plsc.VectorSubcoreMesh(core_axis_name="c", subcore_axis_name="s")


def _dispatch(x, pos):
    @functools.partial(
        pl.kernel,
        mesh=_sc_mesh(),
        out_type=jax.ShapeDtypeStruct((CAP, D), jnp.float32),
        scratch_types=[
            pltpu.VMEM((2 * NSUB, SUB), jnp.int32),
            pltpu.VMEM((SUB, D), jnp.float32),
            pltpu.SemaphoreType.DMA,
        ],
    )
    def k(x_hbm, pos_hbm, xs_hbm, idx_v, rows_v, sem):
        wid = lax.axis_index("s") * 2 + lax.axis_index("c")
        base = wid * TPW
        for kk in range(2):
            for c in range(NSUB):
                pltpu.sync_copy(
                    pos_hbm.at[pl.ds(kk * T + base + c * SUB, SUB)],
                    idx_v.at[kk * NSUB + c],
                )
        for c in range(NSUB):
            pltpu.sync_copy(x_hbm.at[pl.ds(base + c * SUB, SUB)], rows_v)
            cp0 = pltpu.async_copy(rows_v, xs_hbm.at[idx_v.at[c]], sem)
            cp1 = pltpu.async_copy(rows_v, xs_hbm.at[idx_v.at[NSUB + c]], sem)
            cp0.wait()
            cp1.wait()

    return k(x, pos.reshape(2 * T))


def _combine(ys, pos, wp):
    wp_planar = wp.T.reshape(2 * T)

    @functools.partial(
        pl.kernel,
        mesh=_sc_mesh(),
        out_type=jax.ShapeDtypeStruct((T, D), jnp.float32),
        scratch_types=[
            pltpu.VMEM((2 * NSUB, SUB), jnp.int32),
            pltpu.VMEM((SUB, D), jnp.float32),
            pltpu.VMEM((SUB, D), jnp.float32),
            pltpu.VMEM((SUB, D), jnp.float32),
            pltpu.VMEM((2, SUB), jnp.float32),
            pltpu.SemaphoreType.DMA,
        ],
    )
    def k(ys_hbm, pos_hbm, wp_hbm, out_hbm, idx_v, r0_v, r1_v, o_v, w_v, sem):
        wid = lax.axis_index("s") * 2 + lax.axis_index("c")
        base = wid * TPW
        for kk in range(2):
            for c in range(NSUB):
                pltpu.sync_copy(
                    pos_hbm.at[pl.ds(kk * T + base + c * SUB, SUB)],
                    idx_v.at[kk * NSUB + c],
                )
        for c in range(NSUB):
            tb = base + c * SUB
            cp0 = pltpu.async_copy(ys_hbm.at[idx_v.at[c]], r0_v, sem)
            cp1 = pltpu.async_copy(ys_hbm.at[idx_v.at[NSUB + c]], r1_v, sem)
            pltpu.sync_copy(wp_hbm.at[pl.ds(tb, SUB)], w_v.at[0])
            pltpu.sync_copy(wp_hbm.at[pl.ds(T + tb, SUB)], w_v.at[1])
            cp0.wait()
            cp1.wait()
            w0vec = w_v[0, :]
            w1vec = w_v[1, :]
            for r in range(SUB):
                ridx = jnp.full((16,), r, jnp.int32)
                w0 = w0vec.at[ridx].get(mode="promise_in_bounds")
                w1 = w1vec.at[ridx].get(mode="promise_in_bounds")

                def body(j, _, r=r, w0=w0, w1=w1):
                    sl = pl.ds(j * 16, 16)
                    o_v[r, sl] = w0 * r0_v[r, sl] + w1 * r1_v[r, sl]
                    return 0

                lax.fori_loop(0, D // 16, body, 0)
            pltpu.sync_copy(o_v, out_hbm.at[pl.ds(tb, SUB)])

    return k(ys, pos.reshape(2 * T), wp_planar)


def kernel(hidden_states, gate_w, w1, w2):
    logits, wp, pos, be = _router_plan(hidden_states, gate_w)
    be = be.reshape(NBLK)
    xs = _dispatch(hidden_states, pos)
    a = _gmm1(xs, w1, be)
    ys = _gmm2(a, w2, be)
    out = _combine(ys, pos, wp)
    return out, logits


# speedup vs baseline: 1.1151x; 1.1151x over previous
import functools

import jax
import jax.numpy as jnp
from jax import lax
from jax.experimental import pallas as pl
from jax.experimental.pallas import tpu as pltpu
from jax.experimental.pallas import tpu_sc as plsc

T = 2048
D = 2048
FF = 1024
E = 8
TM = 256
CAP = 2 * T + E * TM
NBLK = CAP // TM

_INTERPRET = False


def _router_plan_body(x_ref, gw_ref, logits_ref, wp_ref, pos_ref, be_ref):
    x = x_ref[...]
    gw = gw_ref[...]
    logits = jnp.dot(x, gw, preferred_element_type=jnp.float32)
    logits_ref[...] = logits

    m = jnp.max(logits, axis=1, keepdims=True)
    p = jnp.exp(logits - m)
    p = p / jnp.sum(p, axis=1, keepdims=True)

    lane = jax.lax.broadcasted_iota(jnp.int32, (T, E), 1)
    m0 = jnp.max(p, axis=1, keepdims=True)
    a0 = jnp.min(jnp.where(p == m0, lane, E), axis=1, keepdims=True)
    p1 = jnp.where(lane == a0, -1.0, p)
    m1 = jnp.max(p1, axis=1, keepdims=True)
    a1 = jnp.min(jnp.where(p1 == m1, lane, E), axis=1, keepdims=True)
    s = m0 + m1
    wp_ref[:, 0:1] = m0 / s
    wp_ref[:, 1:2] = m1 / s

    oh0 = (lane == a0).astype(jnp.float32)
    oh1 = (lane == a1).astype(jnp.float32)

    counts = jnp.sum(oh0, axis=0, keepdims=True) + jnp.sum(oh1, axis=0, keepdims=True)
    padded = jnp.floor((counts + (TM - 1)) / TM) * TM
    r8 = jax.lax.broadcasted_iota(jnp.int32, (E, E), 0)
    c8 = jax.lax.broadcasted_iota(jnp.int32, (E, E), 1)
    mstrict = (r8 < c8).astype(jnp.float32)
    offs = jnp.dot(padded, mstrict, preferred_element_type=jnp.float32)

    B = 512
    rb = jax.lax.broadcasted_iota(jnp.int32, (B, B), 0)
    cb = jax.lax.broadcasted_iota(jnp.int32, (B, B), 1)
    ltri = (rb >= cb).astype(jnp.float32)
    carry = jnp.zeros((1, E), jnp.float32)
    excl_parts = []
    for half in (oh0, oh1):
        for b in range(T // B):
            blk = half[b * B:(b + 1) * B, :]
            incl = jnp.dot(ltri, blk, preferred_element_type=jnp.float32) + carry
            excl_parts.append(incl - blk)
            carry = incl[B - 1:B, :]
    oh = jnp.concatenate([oh0, oh1], axis=0)
    excl = jnp.concatenate(excl_parts, axis=0)
    posf = jnp.sum(oh * (offs + excl), axis=1, keepdims=True)
    pos_ref[...] = posf.astype(jnp.int32)

    incl_off = offs + padded
    jv = (jax.lax.broadcasted_iota(jnp.int32, (NBLK, E), 0) * TM).astype(jnp.float32)
    be = jnp.sum((incl_off <= jv).astype(jnp.float32), axis=1, keepdims=True)
    used = jv[:, 0:1] < incl_off[:, E - 1:E]
    be_ref[...] = jnp.where(used, be, -1.0).astype(jnp.int32)


def _router_plan(x, gate_w):
    return pl.pallas_call(
        _router_plan_body,
        out_shape=[
            jax.ShapeDtypeStruct((T, E), jnp.float32),
            jax.ShapeDtypeStruct((T, 2), jnp.float32),
            jax.ShapeDtypeStruct((2 * T, 1), jnp.int32),
            jax.ShapeDtypeStruct((NBLK, 1), jnp.int32),
        ],
        interpret=_INTERPRET,
    )(x, gate_w)


def _gmm_body(be_sref, xs_ref, w1g_ref, w1u_ref, w2_ref, y_ref):
    @pl.when(be_sref[pl.program_id(0)] >= 0)
    def _():
        x = xs_ref[...]
        g = jnp.dot(x, w1g_ref[0], preferred_element_type=jnp.float32)
        u = jnp.dot(x, w1u_ref[0], preferred_element_type=jnp.float32)
        a = g * jax.nn.sigmoid(g) * u
        y_ref[...] = jnp.dot(a, w2_ref[0], preferred_element_type=jnp.float32)


def _gmm(xs, w1, w2, be):
    grid_spec = pltpu.PrefetchScalarGridSpec(
        num_scalar_prefetch=1,
        grid=(NBLK,),
        in_specs=[
            pl.BlockSpec((TM, D), lambda i, be_ref: (i, 0)),
            pl.BlockSpec((1, D, FF), lambda i, be_ref: (jnp.maximum(be_ref[i], 0), 0, 0)),
            pl.BlockSpec((1, D, FF), lambda i, be_ref: (jnp.maximum(be_ref[i], 0), 0, 1)),
            pl.BlockSpec((1, FF, D), lambda i, be_ref: (jnp.maximum(be_ref[i], 0), 0, 0)),
        ],
        out_specs=pl.BlockSpec((TM, D), lambda i, be_ref: (i, 0)),
    )
    return pl.pallas_call(
        _gmm_body,
        grid_spec=grid_spec,
        out_shape=jax.ShapeDtypeStruct((CAP, D), jnp.float32),
        interpret=_INTERPRET,
    )(be, xs, w1, w1, w2)


NW = 32
TPW = T // NW
SUB = 16
NSUB = TPW // SUB


def _sc_mesh():
    return plsc.VectorSubcoreMesh(core_axis_name="c", subcore_axis_name="s")


def _dispatch(x, pos):
    @functools.partial(
        pl.kernel,
        mesh=_sc_mesh(),
        out_type=jax.ShapeDtypeStruct((CAP, D), jnp.float32),
        scratch_types=[
            pltpu.VMEM((2 * NSUB, SUB), jnp.int32),
            pltpu.VMEM((SUB, D), jnp.float32),
            pltpu.SemaphoreType.DMA,
        ],
    )
    def k(x_hbm, pos_hbm, xs_hbm, idx_v, rows_v, sem):
        wid = lax.axis_index("s") * 2 + lax.axis_index("c")
        base = wid * TPW
        for kk in range(2):
            for c in range(NSUB):
                pltpu.sync_copy(
                    pos_hbm.at[pl.ds(kk * T + base + c * SUB, SUB)],
                    idx_v.at[kk * NSUB + c],
                )
        for c in range(NSUB):
            pltpu.sync_copy(x_hbm.at[pl.ds(base + c * SUB, SUB)], rows_v)
            cp0 = pltpu.async_copy(rows_v, xs_hbm.at[idx_v.at[c]], sem)
            cp1 = pltpu.async_copy(rows_v, xs_hbm.at[idx_v.at[NSUB + c]], sem)
            cp0.wait()
            cp1.wait()

    return k(x, pos.reshape(2 * T))


def _combine(ys, pos, wp):
    wp_planar = wp.T.reshape(2 * T)

    @functools.partial(
        pl.kernel,
        mesh=_sc_mesh(),
        out_type=jax.ShapeDtypeStruct((T, D), jnp.float32),
        scratch_types=[
            pltpu.VMEM((2 * NSUB, SUB), jnp.int32),
            pltpu.VMEM((SUB, D), jnp.float32),
            pltpu.VMEM((SUB, D), jnp.float32),
            pltpu.VMEM((SUB, D), jnp.float32),
            pltpu.VMEM((2, SUB), jnp.float32),
            pltpu.SemaphoreType.DMA,
        ],
    )
    def k(ys_hbm, pos_hbm, wp_hbm, out_hbm, idx_v, r0_v, r1_v, o_v, w_v, sem):
        wid = lax.axis_index("s") * 2 + lax.axis_index("c")
        base = wid * TPW
        for kk in range(2):
            for c in range(NSUB):
                pltpu.sync_copy(
                    pos_hbm.at[pl.ds(kk * T + base + c * SUB, SUB)],
                    idx_v.at[kk * NSUB + c],
                )
        for c in range(NSUB):
            tb = base + c * SUB
            cp0 = pltpu.async_copy(ys_hbm.at[idx_v.at[c]], r0_v, sem)
            cp1 = pltpu.async_copy(ys_hbm.at[idx_v.at[NSUB + c]], r1_v, sem)
            pltpu.sync_copy(wp_hbm.at[pl.ds(tb, SUB)], w_v.at[0])
            pltpu.sync_copy(wp_hbm.at[pl.ds(T + tb, SUB)], w_v.at[1])
            cp0.wait()
            cp1.wait()
            w0vec = w_v[0, :]
            w1vec = w_v[1, :]
            for r in range(SUB):
                ridx = jnp.full((16,), r, jnp.int32)
                w0 = w0vec.at[ridx].get(mode="promise_in_bounds")
                w1 = w1vec.at[ridx].get(mode="promise_in_bounds")

                def body(j, _, r=r, w0=w0, w1=w1):
                    sl = pl.ds(j * 16, 16)
                    o_v[r, sl] = w0 * r0_v[r, sl] + w1 * r1_v[r, sl]
                    return 0

                lax.fori_loop(0, D // 16, body, 0)
            pltpu.sync_copy(o_v, out_hbm.at[pl.ds(tb, SUB)])

    return k(ys, pos.reshape(2 * T), wp_planar)


def kernel(hidden_states, gate_w, w1, w2):
    logits, wp, pos, be = _router_plan(hidden_states, gate_w)
    be = be.reshape(NBLK)
    xs = _dispatch(hidden_states, pos)
    ys = _gmm(xs, w1, w2, be)
    out = _combine(ys, pos, wp)
    return out, logits


# speedup vs baseline: 1.2054x; 1.0810x over previous
import functools

import jax
import jax.numpy as jnp
from jax import lax
from jax.experimental import pallas as pl
from jax.experimental.pallas import tpu as pltpu
from jax.experimental.pallas import tpu_sc as plsc

T = 2048
D = 2048
FF = 1024
E = 8
TM = 256
CAP = 2 * T + E * TM
NBLK = CAP // TM

_INTERPRET = False


def _router_plan_body(x_ref, gw_ref, logits_ref, wp_ref, pos_ref, be_ref):
    x = x_ref[...]
    gw = gw_ref[...]
    logits = jnp.dot(x, gw, preferred_element_type=jnp.float32)
    logits_ref[...] = logits

    m = jnp.max(logits, axis=1, keepdims=True)
    p = jnp.exp(logits - m)
    p = p / jnp.sum(p, axis=1, keepdims=True)

    lane = jax.lax.broadcasted_iota(jnp.int32, (T, E), 1)
    m0 = jnp.max(p, axis=1, keepdims=True)
    a0 = jnp.min(jnp.where(p == m0, lane, E), axis=1, keepdims=True)
    p1 = jnp.where(lane == a0, -1.0, p)
    m1 = jnp.max(p1, axis=1, keepdims=True)
    a1 = jnp.min(jnp.where(p1 == m1, lane, E), axis=1, keepdims=True)
    s = m0 + m1
    wp_ref[:, 0:1] = m0 / s
    wp_ref[:, 1:2] = m1 / s

    oh0 = (lane == a0).astype(jnp.float32)
    oh1 = (lane == a1).astype(jnp.float32)

    counts = jnp.sum(oh0, axis=0, keepdims=True) + jnp.sum(oh1, axis=0, keepdims=True)
    padded = jnp.floor((counts + (TM - 1)) / TM) * TM
    r8 = jax.lax.broadcasted_iota(jnp.int32, (E, E), 0)
    c8 = jax.lax.broadcasted_iota(jnp.int32, (E, E), 1)
    mstrict = (r8 < c8).astype(jnp.float32)
    offs = jnp.dot(padded, mstrict, preferred_element_type=jnp.float32)

    B = 512
    rb = jax.lax.broadcasted_iota(jnp.int32, (B, B), 0)
    cb = jax.lax.broadcasted_iota(jnp.int32, (B, B), 1)
    ltri = (rb >= cb).astype(jnp.float32)
    carry = jnp.zeros((1, E), jnp.float32)
    excl_parts = []
    for half in (oh0, oh1):
        for b in range(T // B):
            blk = half[b * B:(b + 1) * B, :]
            incl = jnp.dot(ltri, blk, preferred_element_type=jnp.float32) + carry
            excl_parts.append(incl - blk)
            carry = incl[B - 1:B, :]
    oh = jnp.concatenate([oh0, oh1], axis=0)
    excl = jnp.concatenate(excl_parts, axis=0)
    posf = jnp.sum(oh * (offs + excl), axis=1, keepdims=True)
    pos_ref[...] = posf.astype(jnp.int32)

    incl_off = offs + padded
    jv = (jax.lax.broadcasted_iota(jnp.int32, (NBLK, E), 0) * TM).astype(jnp.float32)
    be = jnp.sum((incl_off <= jv).astype(jnp.float32), axis=1, keepdims=True)
    used = jv[:, 0:1] < incl_off[:, E - 1:E]
    be_ref[...] = jnp.where(used, be, -1.0).astype(jnp.int32)


def _router_plan(x, gate_w):
    return pl.pallas_call(
        _router_plan_body,
        out_shape=[
            jax.ShapeDtypeStruct((T, E), jnp.float32),
            jax.ShapeDtypeStruct((T, 2), jnp.float32),
            jax.ShapeDtypeStruct((2 * T, 1), jnp.int32),
            jax.ShapeDtypeStruct((NBLK, 1), jnp.int32),
        ],
        interpret=_INTERPRET,
    )(x, gate_w)


def _gmm_body(be_sref, xs_ref, w1g_ref, w1u_ref, w2_ref, y_ref):
    @pl.when(be_sref[pl.program_id(0)] >= 0)
    def _():
        x = xs_ref[...]
        g = jnp.dot(x, w1g_ref[0], preferred_element_type=jnp.float32)
        u = jnp.dot(x, w1u_ref[0], preferred_element_type=jnp.float32)
        a = g * jax.nn.sigmoid(g) * u
        y_ref[...] = jnp.dot(a, w2_ref[0], preferred_element_type=jnp.float32)


def _gmm(xs, w1, w2, be):
    grid_spec = pltpu.PrefetchScalarGridSpec(
        num_scalar_prefetch=1,
        grid=(NBLK,),
        in_specs=[
            pl.BlockSpec((TM, D), lambda i, be_ref: (i, 0)),
            pl.BlockSpec((1, D, FF), lambda i, be_ref: (jnp.maximum(be_ref[i], 0), 0, 0)),
            pl.BlockSpec((1, D, FF), lambda i, be_ref: (jnp.maximum(be_ref[i], 0), 0, 1)),
            pl.BlockSpec((1, FF, D), lambda i, be_ref: (jnp.maximum(be_ref[i], 0), 0, 0)),
        ],
        out_specs=pl.BlockSpec((TM, D), lambda i, be_ref: (i, 0)),
    )
    return pl.pallas_call(
        _gmm_body,
        grid_spec=grid_spec,
        out_shape=jax.ShapeDtypeStruct((CAP, D), jnp.float32),
        interpret=_INTERPRET,
    )(be, xs, w1, w1, w2)


NW = 32
TPW = T // NW
SUB = 16
NSUB = TPW // SUB


def _sc_mesh():
    return plsc.VectorSubcoreMesh(core_axis_name="c", subcore_axis_name="s")


def _dispatch(x, pos):
    @functools.partial(
        pl.kernel,
        mesh=_sc_mesh(),
        out_type=jax.ShapeDtypeStruct((CAP, D), jnp.float32),
        scratch_types=[
            pltpu.VMEM((2 * NSUB, SUB), jnp.int32),
            pltpu.VMEM((SUB, D), jnp.float32),
            pltpu.SemaphoreType.DMA,
        ],
    )
    def k(x_hbm, pos_hbm, xs_hbm, idx_v, rows_v, sem):
        wid = lax.axis_index("s") * 2 + lax.axis_index("c")
        base = wid * TPW
        for kk in range(2):
            for c in range(NSUB):
                pltpu.sync_copy(
                    pos_hbm.at[pl.ds(kk * T + base + c * SUB, SUB)],
                    idx_v.at[kk * NSUB + c],
                )
        for c in range(NSUB):
            pltpu.sync_copy(x_hbm.at[pl.ds(base + c * SUB, SUB)], rows_v)
            cp0 = pltpu.async_copy(rows_v, xs_hbm.at[idx_v.at[c]], sem)
            cp1 = pltpu.async_copy(rows_v, xs_hbm.at[idx_v.at[NSUB + c]], sem)
            cp0.wait()
            cp1.wait()

    return k(x, pos.reshape(2 * T))


def _combine(ys, pos, wp):
    wp_planar = wp.T.reshape(2 * T)
    CSUB = 8
    CN = TPW // CSUB

    @functools.partial(
        pl.kernel,
        mesh=_sc_mesh(),
        out_type=jax.ShapeDtypeStruct((T, D), jnp.float32),
        scratch_types=[
            pltpu.VMEM((2 * CN, CSUB), jnp.int32),
            pltpu.VMEM((2, CSUB, D), jnp.float32),
            pltpu.VMEM((2, CSUB, D), jnp.float32),
            pltpu.VMEM((CSUB, D), jnp.float32),
            pltpu.VMEM((2, 16), jnp.float32),
            pltpu.SemaphoreType.DMA,
            pltpu.SemaphoreType.DMA,
        ],
    )
    def k(ys_hbm, pos_hbm, wp_hbm, out_hbm, idx_v, r0_v, r1_v, o_v, w_v, sem0, sem1):
        wid = lax.axis_index("s") * 2 + lax.axis_index("c")
        base = wid * TPW
        for kk in range(2):
            for c in range(CN):
                pltpu.sync_copy(
                    pos_hbm.at[pl.ds(kk * T + base + c * CSUB, CSUB)],
                    idx_v.at[kk * CN + c],
                )
        sems = (sem0, sem1)

        def start(c, b):
            cp0 = pltpu.async_copy(ys_hbm.at[idx_v.at[c]], r0_v.at[b], sems[b])
            cp1 = pltpu.async_copy(ys_hbm.at[idx_v.at[CN + c]], r1_v.at[b], sems[b])
            return cp0, cp1

        pend = start(0, 0)
        for c in range(CN):
            b = c % 2
            nxt = start(c + 1, 1 - b) if c + 1 < CN else None
            tb = base + c * CSUB
            pltpu.sync_copy(wp_hbm.at[pl.ds(tb, CSUB)], w_v.at[0, pl.ds(0, CSUB)])
            pltpu.sync_copy(wp_hbm.at[pl.ds(T + tb, CSUB)], w_v.at[1, pl.ds(0, CSUB)])
            pend[0].wait()
            pend[1].wait()
            w0row = w_v[0, :]
            w1row = w_v[1, :]
            for r in range(CSUB):
                ridx = jnp.full((16,), r, jnp.int32)
                w0 = w0row.at[ridx].get(mode="promise_in_bounds")
                w1 = w1row.at[ridx].get(mode="promise_in_bounds")

                @plsc.parallel_loop(0, D // 16, unroll=4)
                def body(j, r=r, b=b, w0=w0, w1=w1):
                    sl = pl.ds(j * 16, 16)
                    o_v[r, sl] = w0 * r0_v[b, r, sl] + w1 * r1_v[b, r, sl]

            pltpu.sync_copy(o_v, out_hbm.at[pl.ds(tb, CSUB)])
            if nxt is not None:
                pend = nxt

    return k(ys, pos.reshape(2 * T), wp_planar)


def kernel(hidden_states, gate_w, w1, w2):
    logits, wp, pos, be = _router_plan(hidden_states, gate_w)
    be = be.reshape(NBLK)
    xs = _dispatch(hidden_states, pos)
    ys = _gmm(xs, w1, w2, be)
    out = _combine(ys, pos, wp)
    return out, logits
